# Initial kernel scaffold; baseline (speedup 1.0000x reference)
import jax
import jax.numpy as jnp
from jax.experimental import pallas as pl


def kernel(h_by_region, neighbor_indices, W_msg, b_msg, W_merge, b_merge):
    raise NotImplementedError("write your pallas kernel here")


# speedup vs baseline: 1.7799x; 1.7799x over previous
import functools

import jax
import jax.numpy as jnp
from jax import lax
from jax.experimental import pallas as pl
from jax.experimental.pallas import tpu as pltpu
from jax.experimental.pallas import tpu_sc as plsc

R, B, D, K, TOPK = 1024, 32, 128, 16, 4
BD = B * D

RBLK = 128
PBLK = RBLK * B


def _tc_gram_body(hb_ref, hfull_ref, h2_ref, wm_ref, s_ref, p_ref):
    s_ref[...] = lax.dot_general(
        hb_ref[...], hfull_ref[...], (((1,), (1,)), ((), ())),
        precision=lax.Precision.HIGHEST,
        preferred_element_type=jnp.float32)
    wm1 = wm_ref[:, :D]
    p_ref[...] = lax.dot_general(
        h2_ref[...], wm1, (((1,), (1,)), ((), ())),
        preferred_element_type=jnp.float32)


def _tc_gram(h1, h2, w_merge):
    return pl.pallas_call(
        _tc_gram_body,
        grid=(R // RBLK,),
        in_specs=[
            pl.BlockSpec((RBLK, BD), lambda i: (i, 0)),
            pl.BlockSpec((R, BD), lambda i: (0, 0)),
            pl.BlockSpec((PBLK, D), lambda i: (i, 0)),
            pl.BlockSpec((D, 2 * D), lambda i: (0, 0)),
        ],
        out_specs=[
            pl.BlockSpec((RBLK, R), lambda i: (i, 0)),
            pl.BlockSpec((PBLK, D), lambda i: (i, 0)),
        ],
        out_shape=[
            jax.ShapeDtypeStruct((R, R), jnp.float32),
            jax.ShapeDtypeStruct((R * B, D), jnp.float32),
        ],
    )(h1, h1, h2, w_merge)


NW = 32
RW = R // NW
CH = 4
NCHUNK = RW // CH


def _sc_select_gather():
    mesh = plsc.VectorSubcoreMesh(core_axis_name="c", subcore_axis_name="s")

    @functools.partial(
        pl.kernel,
        mesh=mesh,
        out_type=jax.ShapeDtypeStruct((R, BD), jnp.float32),
        compiler_params=pltpu.CompilerParams(use_tc_tiling_on_sc=False,
                                             needs_layout_passes=False),
        scratch_types=[
            pltpu.VMEM((RW, K), jnp.int32),
            pltpu.VMEM((CH * R,), jnp.float32),
            pltpu.VMEM((CH * K,), jnp.int32),
            pltpu.VMEM((CH * TOPK,), jnp.int32),
            pltpu.VMEM((CH * TOPK, BD), jnp.float32),
            pltpu.VMEM((CH, BD), jnp.float32),
            pltpu.SemaphoreType.DMA,
        ],
    )
    def body(s_hbm, nbr_hbm, h_hbm, hbar_hbm,
             nbr_v, s_v, sel_v, idx_v, rows_v, acc_v, sem):
        cid = lax.axis_index("c")
        sid = lax.axis_index("s")
        wid = sid * 2 + cid
        base = wid * RW
        pltpu.sync_copy(nbr_hbm.at[pl.ds(base, RW)], nbr_v)

        lanes = lax.iota(jnp.int32, 16)
        perm = (lanes >> 2) * K + (lanes & 3)

        for c in range(NCHUNK):
            rbase = c * CH
            pltpu.sync_copy(s_hbm.at[pl.ds((base + rbase) * R, CH * R)], s_v)
            for rr in range(CH):
                cand = nbr_v[rbase + rr, :]
                sims = plsc.load_gather(s_v, [cand + (rr * R)])
                _, order = plsc.sort_key_val(sims, cand, descending=True)
                sel_v[pl.ds(rr * K, K)] = order
            idx_v[...] = plsc.load_gather(sel_v, [perm])
            pltpu.async_copy(h_hbm.at[idx_v], rows_v, sem).wait()

            def accum(j, _):
                off = j * 16
                for rr in range(CH):
                    rb = rr * TOPK
                    acc = (rows_v[rb, pl.ds(off, 16)]
                           + rows_v[rb + 1, pl.ds(off, 16)]
                           + rows_v[rb + 2, pl.ds(off, 16)]
                           + rows_v[rb + 3, pl.ds(off, 16)])
                    acc_v[rr, pl.ds(off, 16)] = acc * 0.25
                return 0
            lax.fori_loop(0, BD // 16, accum, 0)
            pltpu.sync_copy(acc_v, hbar_hbm.at[pl.ds(base + rbase, CH)])

    return body


CBLK = 4096


def _tc_merge_body(p_ref, hbar_ref, wm_ref, wmsg_ref, bmsg_ref, bmrg_ref,
                   out_ref):
    wm2 = wm_ref[:, D:]
    w_eff = lax.dot_general(wm2, wmsg_ref[...], (((1,), (0,)), ((), ())),
                            preferred_element_type=jnp.float32)
    b_eff = bmrg_ref[...] + lax.dot_general(
        bmsg_ref[...], wm2, (((1,), (1,)), ((), ())),
        preferred_element_type=jnp.float32)
    out_ref[...] = (p_ref[...]
                    + lax.dot_general(hbar_ref[...], w_eff,
                                      (((1,), (1,)), ((), ())),
                                      preferred_element_type=jnp.float32)
                    + b_eff)


def _tc_merge(p, hbar2, w_merge, w_msg, b_msg, b_merge):
    return pl.pallas_call(
        _tc_merge_body,
        grid=(R * B // CBLK,),
        in_specs=[
            pl.BlockSpec((CBLK, D), lambda i: (i, 0)),
            pl.BlockSpec((CBLK, D), lambda i: (i, 0)),
            pl.BlockSpec((D, 2 * D), lambda i: (0, 0)),
            pl.BlockSpec((D, D), lambda i: (0, 0)),
            pl.BlockSpec((1, D), lambda i: (0, 0)),
            pl.BlockSpec((1, D), lambda i: (0, 0)),
        ],
        out_specs=pl.BlockSpec((CBLK, D), lambda i: (i, 0)),
        out_shape=jax.ShapeDtypeStruct((R * B, D), jnp.float32),
    )(p, hbar2, w_merge, w_msg, b_msg, b_merge)


def kernel(h_by_region, neighbor_indices, W_msg, b_msg, W_merge, b_merge):
    h1 = h_by_region.reshape(R, BD)
    h2 = h_by_region.reshape(R * B, D)
    s, p = _tc_gram(h1, h2, W_merge)
    hbar = _sc_select_gather()(s.reshape(R * R), neighbor_indices, h1)
    out = _tc_merge(p, hbar.reshape(R * B, D), W_merge, W_msg,
                    b_msg.reshape(1, D), b_merge.reshape(1, D))
    return out.reshape(R, B, D)


# speedup vs baseline: 2.4305x; 1.3655x over previous
import functools

import jax
import jax.numpy as jnp
from jax import lax
from jax.experimental import pallas as pl
from jax.experimental.pallas import tpu as pltpu
from jax.experimental.pallas import tpu_sc as plsc

R, B, D, K, TOPK = 1024, 32, 128, 16, 4
BD = B * D

RBLK = 128


def _tc_gram_body(hfull_ref, s_ref, hhi_ref, hlo_ref):
    i = pl.program_id(0)

    @pl.when(i == 0)
    def _split():
        hf = hfull_ref[...]
        hi = hf.astype(jnp.bfloat16)
        hhi_ref[...] = hi
        hlo_ref[...] = (hf - hi.astype(jnp.float32)).astype(jnp.bfloat16)

    lhs_hi = hhi_ref[pl.ds(i * RBLK, RBLK), :]
    lhs_lo = hlo_ref[pl.ds(i * RBLK, RBLK), :]
    rhs_hi = hhi_ref[...]
    rhs_lo = hlo_ref[...]
    dn = (((1,), (1,)), ((), ()))
    s_ref[...] = (
        lax.dot_general(lhs_hi, rhs_hi, dn,
                        preferred_element_type=jnp.float32)
        + lax.dot_general(lhs_hi, rhs_lo, dn,
                          preferred_element_type=jnp.float32)
        + lax.dot_general(lhs_lo, rhs_hi, dn,
                          preferred_element_type=jnp.float32))


def _tc_gram(h1):
    return pl.pallas_call(
        _tc_gram_body,
        grid=(R // RBLK,),
        in_specs=[
            pl.BlockSpec((R, BD), lambda i: (0, 0)),
        ],
        out_specs=pl.BlockSpec((RBLK, R), lambda i: (i, 0)),
        out_shape=jax.ShapeDtypeStruct((R, R), jnp.float32),
        scratch_shapes=[
            pltpu.VMEM((R, BD), jnp.bfloat16),
            pltpu.VMEM((R, BD), jnp.bfloat16),
        ],
    )(h1)


NW = 32
RW = R // NW
CH = 4
NCHUNK = RW // CH


def _sc_select_gather():
    mesh = plsc.VectorSubcoreMesh(core_axis_name="c", subcore_axis_name="s")

    @functools.partial(
        pl.kernel,
        mesh=mesh,
        out_type=jax.ShapeDtypeStruct((R, BD), jnp.float32),
        compiler_params=pltpu.CompilerParams(use_tc_tiling_on_sc=False,
                                             needs_layout_passes=False),
        scratch_types=[
            pltpu.VMEM((RW, K), jnp.int32),
            pltpu.VMEM((CH * R,), jnp.float32),
            pltpu.VMEM((CH * K,), jnp.int32),
            pltpu.VMEM((CH * TOPK,), jnp.int32),
            pltpu.VMEM((CH * TOPK, BD), jnp.float32),
            pltpu.VMEM((CH, BD), jnp.float32),
            pltpu.SemaphoreType.DMA,
        ],
    )
    def body(s_hbm, nbr_hbm, h_hbm, hbar_hbm,
             nbr_v, s_v, sel_v, idx_v, rows_v, acc_v, sem):
        cid = lax.axis_index("c")
        sid = lax.axis_index("s")
        wid = sid * 2 + cid
        base = wid * RW
        pltpu.sync_copy(nbr_hbm.at[pl.ds(base, RW)], nbr_v)

        lanes = lax.iota(jnp.int32, 16)
        perm = (lanes >> 2) * K + (lanes & 3)

        for c in range(NCHUNK):
            rbase = c * CH
            pltpu.sync_copy(s_hbm.at[pl.ds((base + rbase) * R, CH * R)], s_v)
            for rr in range(CH):
                cand = nbr_v[rbase + rr, :]
                sims = plsc.load_gather(s_v, [cand + (rr * R)])
                _, order = plsc.sort_key_val(sims, cand, descending=True)
                sel_v[pl.ds(rr * K, K)] = order
            idx_v[...] = plsc.load_gather(sel_v, [perm])
            pltpu.async_copy(h_hbm.at[idx_v], rows_v, sem).wait()

            def accum(j, _):
                off = j * 16
                for rr in range(CH):
                    rb = rr * TOPK
                    acc = (rows_v[rb, pl.ds(off, 16)]
                           + rows_v[rb + 1, pl.ds(off, 16)]
                           + rows_v[rb + 2, pl.ds(off, 16)]
                           + rows_v[rb + 3, pl.ds(off, 16)])
                    acc_v[rr, pl.ds(off, 16)] = acc * 0.25
                return 0
            lax.fori_loop(0, BD // 16, accum, 0)
            pltpu.sync_copy(acc_v, hbar_hbm.at[pl.ds(base + rbase, CH)])

    return body


CBLK = 4096


def _tc_merge_body(h2_ref, hbar_ref, wm_ref, wmsg_ref, bmsg_ref, bmrg_ref,
                   out_ref):
    wm1 = wm_ref[:, :D]
    wm2 = wm_ref[:, D:]
    w_eff = lax.dot_general(wm2, wmsg_ref[...], (((1,), (0,)), ((), ())),
                            preferred_element_type=jnp.float32)
    b_eff = bmrg_ref[...] + lax.dot_general(
        bmsg_ref[...], wm2, (((1,), (1,)), ((), ())),
        preferred_element_type=jnp.float32)
    dn = (((1,), (1,)), ((), ()))
    out_ref[...] = (lax.dot_general(h2_ref[...], wm1, dn,
                                    preferred_element_type=jnp.float32)
                    + lax.dot_general(hbar_ref[...], w_eff, dn,
                                      preferred_element_type=jnp.float32)
                    + b_eff)


def _tc_merge(h2, hbar2, w_merge, w_msg, b_msg, b_merge):
    return pl.pallas_call(
        _tc_merge_body,
        grid=(R * B // CBLK,),
        in_specs=[
            pl.BlockSpec((CBLK, D), lambda i: (i, 0)),
            pl.BlockSpec((CBLK, D), lambda i: (i, 0)),
            pl.BlockSpec((D, 2 * D), lambda i: (0, 0)),
            pl.BlockSpec((D, D), lambda i: (0, 0)),
            pl.BlockSpec((1, D), lambda i: (0, 0)),
            pl.BlockSpec((1, D), lambda i: (0, 0)),
        ],
        out_specs=pl.BlockSpec((CBLK, D), lambda i: (i, 0)),
        out_shape=jax.ShapeDtypeStruct((R * B, D), jnp.float32),
    )(h2, hbar2, w_merge, w_msg, b_msg, b_merge)


def kernel(h_by_region, neighbor_indices, W_msg, b_msg, W_merge, b_merge):
    h1 = h_by_region.reshape(R, BD)
    h2 = h_by_region.reshape(R * B, D)
    s = _tc_gram(h1)
    hbar = _sc_select_gather()(s.reshape(R * R), neighbor_indices, h1)
    out = _tc_merge(h2, hbar.reshape(R * B, D), W_merge, W_msg,
                    b_msg.reshape(1, D), b_merge.reshape(1, D))
    return out.reshape(R, B, D)
